# baseline (speedup 1.0000x reference)
import jax
import jax.numpy as jnp
from jax import lax
from jax.experimental import pallas as pl
from jax.experimental.pallas import tpu as pltpu
from jax.experimental.pallas import tpu_sc as plsc

_NUM_CORES = 2
_NUM_SUBCORES = 16
_LANES = 16
_NUM_WORKERS = _NUM_CORES * _NUM_SUBCORES

_B, _S, _D = 4, 4096, 1024
_ROWS_PER_W = _S // _NUM_WORKERS
_R = 16
_NTILES = _ROWS_PER_W // _R
_NSTEPS = _NTILES * _B
_NBIO = 4
_NBT = 2


def _add_tile(t_ref, xio_ref):
    @plsc.parallel_loop(0, _R * _D, step=_LANES, unroll=8)
    def _(i):
        r = i >> 10
        c = pl.multiple_of(i & (_D - 1), _LANES)
        plsc.addupdate(xio_ref.at[r, pl.ds(c, _LANES)],
                       t_ref[r, pl.ds(c, _LANES)])


def _sc_body(x_hbm, t_hbm, o_hbm,
             io0, io1, io2, io3, tb0, tb1,
             li0, li1, li2, li3, so0, so1, so2, so3, ts0, ts1):
    wid = lax.axis_index("s") * _NUM_CORES + lax.axis_index("c")
    row0 = wid * _ROWS_PER_W

    xio, tbuf = [io0, io1, io2, io3], [tb0, tb1]
    lsem, ssem, tsem = [li0, li1, li2, li3], [so0, so1, so2, so3], [ts0, ts1]

    def t_load(j):
        return pltpu.async_copy(
            t_hbm.at[pl.ds(row0 + j * _R, _R)],
            tbuf[j % _NBT], tsem[j % _NBT])

    def x_load(s):
        j, b = s // _B, s % _B
        return pltpu.async_copy(
            x_hbm.at[b, pl.ds(row0 + j * _R, _R)],
            xio[s % _NBIO], lsem[s % _NBIO])

    def x_store(s):
        j, b = s // _B, s % _B
        return pltpu.async_copy(
            xio[s % _NBIO],
            o_hbm.at[b, pl.ds(row0 + j * _R, _R)], ssem[s % _NBIO])

    tdesc = {0: t_load(0), 1: t_load(1)}
    xdesc = {0: x_load(0), 1: x_load(1)}
    sdesc = {}

    for s in range(_NSTEPS):
        j, b = s // _B, s % _B
        if b == 0:
            tdesc[j].wait()
        xdesc[s].wait()
        _add_tile(tbuf[j % _NBT], xio[s % _NBIO])
        sdesc[s] = x_store(s)
        if s + 2 < _NSTEPS:
            if s - 2 in sdesc:
                sdesc[s - 2].wait()
            xdesc[s + 2] = x_load(s + 2)
        if b == _B - 1 and j + _NBT < _NTILES:
            tdesc[j + _NBT] = t_load(j + _NBT)

    for s in range(_NSTEPS - 4, _NSTEPS):
        sdesc[s].wait()


def kernel(inputs, pos_table):
    B, S, D = inputs.shape

    mesh = plsc.VectorSubcoreMesh(
        core_axis_name="c", subcore_axis_name="s",
        num_cores=_NUM_CORES, num_subcores=_NUM_SUBCORES,
    )
    return pl.kernel(
        _sc_body,
        out_type=jax.ShapeDtypeStruct((B, S, D), jnp.float32),
        mesh=mesh,
        compiler_params=pltpu.CompilerParams(use_tc_tiling_on_sc=True),
        scratch_types=(
            [pltpu.VMEM((_R, _D), jnp.float32)] * (_NBIO + _NBT)
            + [pltpu.SemaphoreType.DMA] * (_NBIO + _NBIO + _NBT)
        ),
    )(inputs, pos_table)


# speedup vs baseline: 1.0473x; 1.0473x over previous
import jax
import jax.numpy as jnp
from jax import lax
from jax.experimental import pallas as pl
from jax.experimental.pallas import tpu as pltpu
from jax.experimental.pallas import tpu_sc as plsc

_NUM_CORES = 2
_NUM_SUBCORES = 16
_LANES = 16
_NUM_WORKERS = _NUM_CORES * _NUM_SUBCORES

_B, _S, _D = 4, 4096, 1024
_ROWS_PER_W = _S // _NUM_WORKERS
_R = 16
_NTILES = _ROWS_PER_W // _R
_NSTEPS = _NTILES * _B
_NBIN = 3
_NBOUT = 2
_NBT = 2


def _add_tile(xi_ref, t_ref, xo_ref):
    @plsc.parallel_loop(0, _R * _D, step=_LANES, unroll=8)
    def _(i):
        r = i >> 10
        c = pl.multiple_of(i & (_D - 1), _LANES)
        xo_ref[r, pl.ds(c, _LANES)] = (
            xi_ref[r, pl.ds(c, _LANES)] + t_ref[r, pl.ds(c, _LANES)]
        )


def _sc_body(x_hbm, t_hbm, o_hbm,
             xi0, xi1, xi2, xo0, xo1, tb0, tb1,
             li0, li1, li2, so0, so1, ts0, ts1):
    wid = lax.axis_index("s") * _NUM_CORES + lax.axis_index("c")
    row0 = wid * _ROWS_PER_W

    xin, xout, tbuf = [xi0, xi1, xi2], [xo0, xo1], [tb0, tb1]
    lsem, ssem, tsem = [li0, li1, li2], [so0, so1], [ts0, ts1]

    def t_load(j):
        return pltpu.async_copy(
            t_hbm.at[pl.ds(row0 + j * _R, _R)],
            tbuf[j % _NBT], tsem[j % _NBT])

    def x_load(s):
        j, b = s // _B, s % _B
        return pltpu.async_copy(
            x_hbm.at[b, pl.ds(row0 + j * _R, _R)],
            xin[s % _NBIN], lsem[s % _NBIN])

    def x_store(s):
        j, b = s // _B, s % _B
        return pltpu.async_copy(
            xout[s % _NBOUT],
            o_hbm.at[b, pl.ds(row0 + j * _R, _R)], ssem[s % _NBOUT])

    tdesc = {0: t_load(0), 1: t_load(1)}
    xdesc = {s: x_load(s) for s in range(_NBIN)}
    sdesc = {}

    for s in range(_NSTEPS):
        j, b = s // _B, s % _B
        if s - _NBOUT in sdesc:
            sdesc[s - _NBOUT].wait()
        if b == 0:
            tdesc[j].wait()
        xdesc[s].wait()
        _add_tile(xin[s % _NBIN], tbuf[j % _NBT], xout[s % _NBOUT])
        sdesc[s] = x_store(s)
        if s + _NBIN < _NSTEPS:
            xdesc[s + _NBIN] = x_load(s + _NBIN)
        if b == _B - 1 and j + _NBT < _NTILES:
            tdesc[j + _NBT] = t_load(j + _NBT)

    for s in range(_NSTEPS - _NBOUT, _NSTEPS):
        sdesc[s].wait()


def kernel(inputs, pos_table):
    B, S, D = inputs.shape

    mesh = plsc.VectorSubcoreMesh(
        core_axis_name="c", subcore_axis_name="s",
        num_cores=_NUM_CORES, num_subcores=_NUM_SUBCORES,
    )
    return pl.kernel(
        _sc_body,
        out_type=jax.ShapeDtypeStruct((B, S, D), jnp.float32),
        mesh=mesh,
        compiler_params=pltpu.CompilerParams(
            use_tc_tiling_on_sc=True,
            skip_device_barrier=True,
            disable_bounds_checks=True,
            disable_semaphore_checks=True,
        ),
        scratch_types=(
            [pltpu.VMEM((_R, _D), jnp.float32)] * (_NBIN + _NBOUT + _NBT)
            + [pltpu.SemaphoreType.DMA] * (_NBIN + _NBOUT + _NBT)
        ),
    )(inputs, pos_table)
